# Initial kernel scaffold; baseline (speedup 1.0000x reference)
import jax
import jax.numpy as jnp
from jax.experimental import pallas as pl


def kernel(indices, table):
    raise NotImplementedError("write your pallas kernel here")


# speedup vs baseline: 2.9910x; 2.9910x over previous
import functools

import jax
import jax.numpy as jnp
from jax import lax
from jax.experimental import pallas as pl
from jax.experimental.pallas import tpu as pltpu
from jax.experimental.pallas import tpu_sc as plsc

_DP = 32
_L = 128
_G = 8

_INFO = plsc.get_sparse_core_info()
_NC = _INFO.num_cores
_NS = _INFO.num_subcores
_NW = _NC * _NS


@functools.cache
def _build(n_rows):
  rows_per_w = n_rows // _NW
  ng = rows_per_w // _G
  mesh = plsc.VectorSubcoreMesh(core_axis_name="c", subcore_axis_name="s")

  @functools.partial(
      pl.kernel, mesh=mesh,
      out_type=jax.ShapeDtypeStruct((n_rows, _L, _DP), jnp.float32),
      compiler_params=pltpu.CompilerParams(use_tc_tiling_on_sc=False),
      scratch_types=[
          pltpu.VMEM((_G, _L), jnp.int32),
          pltpu.VMEM((_G, _L, _DP), jnp.float32),
          pltpu.SemaphoreType.DMA,
      ],
  )
  def gather_kernel(idx_hbm, table_hbm, out_hbm, idx_v, rows_v, sem):
    wid = lax.axis_index("s") * _NC + lax.axis_index("c")
    base = wid * rows_per_w

    def step(g, carry):
      r0 = base + g * _G
      pltpu.sync_copy(idx_hbm.at[pl.ds(r0, _G)], idx_v)
      copies = [
          pltpu.async_copy(table_hbm.at[idx_v.at[j]], rows_v.at[j], sem)
          for j in range(_G)
      ]
      for c in copies:
        c.wait()
      pltpu.sync_copy(rows_v, out_hbm.at[pl.ds(r0, _G)])
      return carry

    lax.fori_loop(0, ng, step, 0)

  return gather_kernel


def kernel(indices, table):
  B, S = indices.shape
  V, D = table.shape
  n = B * S
  idx = indices.reshape(n // _L, _L).astype(jnp.int32)
  tab = jnp.pad(table, ((0, 0), (0, _DP - D)))
  out = _build(n // _L)(idx, tab)
  return out[:, :, :D].reshape(B, S, D)
